# baseline (speedup 1.0000x reference)
import functools

import jax
import jax.numpy as jnp
from jax import lax
from jax.experimental import pallas as pl
from jax.experimental.pallas import tpu as pltpu
from jax.experimental.pallas import tpu_sc as plsc

_EMB = 64
_B = 16384
_TITLE_LEN = 20
_DESC_LEN = 200

_NC = 2
_NS = 16
_NW = _NC * _NS

_IW = 128
_KPC = 4
_CHUNK = _KPC * _IW

_T_TOTAL = _B * _TITLE_LEN
_D_TOTAL = _B * _DESC_LEN
_T_ROWS = _T_TOTAL // _IW
_D_ROWS = _D_TOTAL // _IW
_T_ROWS_W = _T_ROWS // _NW
_D_ROWS_W = _D_ROWS // _NW


def _gather_table(tbl_hbm, idx_hbm, out_hbm, idx_v, rows_v, sem, wid, rows_w):
    base = wid * rows_w
    n_chunks = rows_w // _KPC

    @pl.loop(0, n_chunks)
    def _chunk(i):
        row_off = base + i * _KPC
        pltpu.sync_copy(idx_hbm.at[pl.ds(row_off, _KPC)], idx_v)
        descs = [
            pltpu.async_copy(
                tbl_hbm.at[idx_v.at[j]],
                rows_v.at[pl.ds(j * _IW, _IW)],
                sem,
            )
            for j in range(_KPC)
        ]
        for d in descs:
            d.wait()
        pltpu.sync_copy(rows_v, out_hbm.at[pl.ds(row_off * _IW, _CHUNK)])


def _body(t_idx, d_idx, t_tbl, d_tbl, out_t, out_d, idx_v, rows_v, sem):
    wid = lax.axis_index("s") * _NC + lax.axis_index("c")
    _gather_table(t_tbl, t_idx, out_t, idx_v, rows_v, sem, wid, _T_ROWS_W)
    _gather_table(d_tbl, d_idx, out_d, idx_v, rows_v, sem, wid, _D_ROWS_W)


@jax.jit
def _lookup(t_idx, d_idx, t_tbl, d_tbl):
    mesh = plsc.VectorSubcoreMesh(core_axis_name="c", subcore_axis_name="s")
    return pl.kernel(
        _body,
        out_type=(
            jax.ShapeDtypeStruct((_T_TOTAL, _EMB), jnp.float32),
            jax.ShapeDtypeStruct((_D_TOTAL, _EMB), jnp.float32),
        ),
        mesh=mesh,
        scratch_types=[
            pltpu.VMEM((_KPC, _IW), jnp.int32),
            pltpu.VMEM((_CHUNK, _EMB), jnp.float32),
            pltpu.SemaphoreType.DMA,
        ],
        compiler_params=pltpu.CompilerParams(use_tc_tiling_on_sc=False),
    )(t_idx, d_idx, t_tbl, d_tbl)


def kernel(title, description, title_table, description_table):
    t_idx = title.reshape(_T_ROWS, _IW).astype(jnp.int32)
    d_idx = description.reshape(_D_ROWS, _IW).astype(jnp.int32)
    out_t, out_d = _lookup(t_idx, d_idx, title_table, description_table)
    return (
        out_t.reshape(_B, _TITLE_LEN, _EMB),
        out_d.reshape(_B, _DESC_LEN, _EMB),
    )


# speedup vs baseline: 1.0631x; 1.0631x over previous
import functools

import jax
import jax.numpy as jnp
from jax import lax
from jax.experimental import pallas as pl
from jax.experimental.pallas import tpu as pltpu
from jax.experimental.pallas import tpu_sc as plsc

_EMB = 64
_B = 16384
_TITLE_LEN = 20
_DESC_LEN = 200

_NC = 2
_NS = 16
_NW = _NC * _NS

_IW = 128
_KPC = 4
_CHUNK = _KPC * _IW
_NBUF = 2

_T_TOTAL = _B * _TITLE_LEN
_D_TOTAL = _B * _DESC_LEN
_T_ROWS = _T_TOTAL // _IW
_D_ROWS = _D_TOTAL // _IW
_T_ROWS_W = _T_ROWS // _NW
_D_ROWS_W = _D_ROWS // _NW


def _gather_table(tbl, idx_hbm, out_hbm, idx_v, rows_v, sem_i, sem_g, sem_w,
                  wid, rows_w):
    base = wid * rows_w
    n_chunks = rows_w // _KPC
    n_groups = n_chunks // _NBUF

    def idx_src(c):
        return idx_hbm.at[pl.ds(base + c * _KPC, _KPC)]

    def out_dst(c):
        return out_hbm.at[pl.ds((base + c * _KPC) * _IW, _CHUNK)]

    def fire_gathers(b):
        for j in range(_KPC):
            pltpu.async_copy(
                tbl.at[idx_v.at[b, j]],
                rows_v.at[b, pl.ds(j * _IW, _IW)],
                sem_g[b],
            )

    def wait_gathers(b):
        for j in range(_KPC):
            pltpu.make_async_copy(
                tbl.at[idx_v.at[b, j]],
                rows_v.at[b, pl.ds(j * _IW, _IW)],
                sem_g[b],
            ).wait()

    def wait_idx(b):
        pltpu.make_async_copy(idx_src(0), idx_v.at[b], sem_i[b]).wait()

    def wait_wb(b):
        pltpu.make_async_copy(rows_v.at[b], out_dst(0), sem_w[b]).wait()

    for b in range(_NBUF):
        pltpu.async_copy(idx_src(b), idx_v.at[b], sem_i[b])

    @pl.loop(0, n_groups)
    def _group(gi):
        c0 = gi * _NBUF
        for b in range(_NBUF):
            wait_idx(b)

            @pl.when(gi > 0)
            def _():
                wait_wb(b)

            fire_gathers(b)
        for b in range(_NBUF):
            wait_gathers(b)
            pltpu.async_copy(rows_v.at[b], out_dst(c0 + b), sem_w[b])

            @pl.when(gi < n_groups - 1)
            def _():
                pltpu.async_copy(idx_src(c0 + _NBUF + b), idx_v.at[b], sem_i[b])

    for b in range(_NBUF):
        wait_wb(b)


def _body(t_idx, d_idx, t_tbl, d_tbl, out_t, out_d, idx_v, rows_v,
          si0, si1, sg0, sg1, sw0, sw1):
    wid = lax.axis_index("s") * _NC + lax.axis_index("c")
    sem_i = (si0, si1)
    sem_g = (sg0, sg1)
    sem_w = (sw0, sw1)
    _gather_table(t_tbl, t_idx, out_t, idx_v, rows_v, sem_i, sem_g, sem_w,
                  wid, _T_ROWS_W)
    _gather_table(d_tbl, d_idx, out_d, idx_v, rows_v, sem_i, sem_g, sem_w,
                  wid, _D_ROWS_W)


@jax.jit
def _lookup(t_idx, d_idx, t_tbl, d_tbl):
    mesh = plsc.VectorSubcoreMesh(core_axis_name="c", subcore_axis_name="s")
    return pl.kernel(
        _body,
        out_type=(
            jax.ShapeDtypeStruct((_T_TOTAL, _EMB), jnp.float32),
            jax.ShapeDtypeStruct((_D_TOTAL, _EMB), jnp.float32),
        ),
        mesh=mesh,
        scratch_types=[
            pltpu.VMEM((_NBUF, _KPC, _IW), jnp.int32),
            pltpu.VMEM((_NBUF, _CHUNK, _EMB), jnp.float32),
        ] + [pltpu.SemaphoreType.DMA] * 6,
        compiler_params=pltpu.CompilerParams(use_tc_tiling_on_sc=False),
    )(t_idx, d_idx, t_tbl, d_tbl)


def kernel(title, description, title_table, description_table):
    t_idx = title.reshape(_T_ROWS, _IW).astype(jnp.int32)
    d_idx = description.reshape(_D_ROWS, _IW).astype(jnp.int32)
    out_t, out_d = _lookup(t_idx, d_idx, title_table, description_table)
    return (
        out_t.reshape(_B, _TITLE_LEN, _EMB),
        out_d.reshape(_B, _DESC_LEN, _EMB),
    )
